# baseline (speedup 1.0000x reference)
import functools

import jax
import jax.numpy as jnp
from jax import lax
from jax.experimental import pallas as pl
from jax.experimental.pallas import tpu as pltpu
from jax.experimental.pallas import tpu_sc as plsc

_N = 131072
_CIN = 64
_RES = 256
_NB = 16
_NCLS = 20

_ROWS = 128
_COLS = 1024
_BR = 8
_BN = _BR * _COLS
_GRID1 = _ROWS // _BR

_NW = 32
_PPW = _N // _NW
_WIN = 2048
_NCH = _WIN // 128

_CELLS = _RES * _RES
_CLSZ = _CELLS * _NCLS
_MSKZ = _CELLS * 2
_CPS = _CLSZ // 16
_MPS = _MSKZ // 16
_ZB = 2048


def _k1_body(t_s, bx_s, bm_s, bf_s, x_r, y_r, z_r, ft_r, w2_r,
             xt_o, yt_o, as_o, in_o, mf_o, mm_o):
    i = pl.program_id(0)
    f32 = jnp.float32
    bf16 = jnp.bfloat16
    x = x_r[...].astype(bf16).astype(f32)
    y = y_r[...].astype(bf16).astype(f32)
    z = z_r[...].astype(bf16).astype(f32)

    def _b(v):
        return v.astype(bf16).astype(f32)

    xt = x * _b(t_s[0, 0]) + y * _b(t_s[0, 1]) + z * _b(t_s[0, 2]) + t_s[0, 3]
    yt = x * _b(t_s[1, 0]) + y * _b(t_s[1, 1]) + z * _b(t_s[1, 2]) + t_s[1, 3]
    zt = x * _b(t_s[2, 0]) + y * _b(t_s[2, 1]) + z * _b(t_s[2, 2]) + t_s[2, 3]

    best = None
    asn = None
    ins = None
    for j in range(_NB):
        dx = xt - bx_s[j, 0]
        dy = yt - bx_s[j, 1]
        dz = zt - bx_s[j, 2]
        d = jnp.sqrt(dx * dx + dy * dy + dz * dz)
        hx = jnp.abs(bx_s[j, 3]) + 1e-3
        hy = jnp.abs(bx_s[j, 4]) + 1e-3
        hz = jnp.abs(bx_s[j, 5]) + 1e-3
        inj = jnp.where(
            (jnp.abs(dx) <= hx) & (jnp.abs(dy) <= hy) & (jnp.abs(dz) <= hz),
            jnp.float32(1.0), jnp.float32(0.0))
        if j == 0:
            best = d
            asn = jnp.zeros(d.shape, jnp.int32)
            ins = inj
        else:
            upd = d < best
            best = jnp.where(upd, d, best)
            asn = jnp.where(upd, j, asn)
            ins = jnp.where(upd, inj, ins)
    xt_o[...] = xt
    yt_o[...] = yt
    as_o[...] = asn
    in_o[...] = ins

    fb = ft_r[...].astype(bf16)
    res = lax.dot_general(w2_r[...].astype(bf16), fb, (((1,), (1,)), ((), ())),
                          preferred_element_type=jnp.float32)
    a0 = jax.nn.sigmoid(res[0:1, :] + bm_s[0])
    a1 = jax.nn.sigmoid(res[1:2, :] + bm_s[1])
    mf_o[0:1, :] = (res[2:3, :] + bf_s[0]) * a0
    mf_o[1:2, :] = (res[3:4, :] + bf_s[1]) * a1

    @pl.when(i == 0)
    def _init():
        mm_o[...] = jnp.zeros((8, 128), jnp.float32)
        mm_o[0:2, :] = jnp.full((2, 128), jnp.inf, jnp.float32)
        mm_o[2:4, :] = jnp.full((2, 128), -jnp.inf, jnp.float32)

    mm_o[0:1, :] = jnp.minimum(mm_o[0:1, :], jnp.min(xt))
    mm_o[1:2, :] = jnp.minimum(mm_o[1:2, :], jnp.min(yt))
    mm_o[2:3, :] = jnp.maximum(mm_o[2:3, :], jnp.max(xt))
    mm_o[3:4, :] = jnp.maximum(mm_o[3:4, :], jnp.max(yt))


def _k1(transform, boxes, b_matte, b_feat, x2, y2, z2, ftT, w2):
    f32 = jnp.float32
    return pl.pallas_call(
        _k1_body,
        grid=(_GRID1,),
        in_specs=[
            pl.BlockSpec(memory_space=pltpu.SMEM),
            pl.BlockSpec(memory_space=pltpu.SMEM),
            pl.BlockSpec(memory_space=pltpu.SMEM),
            pl.BlockSpec(memory_space=pltpu.SMEM),
            pl.BlockSpec((_BR, _COLS), lambda i: (i, 0)),
            pl.BlockSpec((_BR, _COLS), lambda i: (i, 0)),
            pl.BlockSpec((_BR, _COLS), lambda i: (i, 0)),
            pl.BlockSpec((_BN, _CIN), lambda i: (i, 0)),
            pl.BlockSpec((8, _CIN), lambda i: (0, 0)),
        ],
        out_specs=[
            pl.BlockSpec((_BR, _COLS), lambda i: (i, 0)),
            pl.BlockSpec((_BR, _COLS), lambda i: (i, 0)),
            pl.BlockSpec((_BR, _COLS), lambda i: (i, 0)),
            pl.BlockSpec((_BR, _COLS), lambda i: (i, 0)),
            pl.BlockSpec((2, _BN), lambda i: (0, i)),
            pl.BlockSpec((8, 128), lambda i: (0, 0)),
        ],
        out_shape=[
            jax.ShapeDtypeStruct((_ROWS, _COLS), f32),
            jax.ShapeDtypeStruct((_ROWS, _COLS), f32),
            jax.ShapeDtypeStruct((_ROWS, _COLS), jnp.int32),
            jax.ShapeDtypeStruct((_ROWS, _COLS), f32),
            jax.ShapeDtypeStruct((2, _N), f32),
            jax.ShapeDtypeStruct((8, 128), f32),
        ],
    )(transform, boxes, b_matte, b_feat, x2, y2, z2, ftT, w2)


def _sc_mesh():
    return plsc.VectorSubcoreMesh(core_axis_name="c", subcore_axis_name="s")


_SC_PARAMS = pltpu.CompilerParams(needs_layout_passes=False)


def _k2c(xt, yt, asn, ins, pc, mf, mmf):
    f32 = jnp.float32
    i32 = jnp.int32
    _HB = _NCLS * _NB
    _HPW = _N // 16

    @functools.partial(
        pl.kernel,
        mesh=_sc_mesh(),
        out_type=[
            jax.ShapeDtypeStruct((2, _CLSZ), f32),
            jax.ShapeDtypeStruct((2, _MSKZ), f32),
        ],
        compiler_params=_SC_PARAMS,
        scratch_types=[
            pltpu.VMEM((_WIN,), f32),
            pltpu.VMEM((_WIN,), f32),
            pltpu.VMEM((_WIN,), i32),
            pltpu.VMEM((_WIN,), f32),
            pltpu.VMEM((_WIN,), i32),
            pltpu.VMEM((_WIN,), f32),
            pltpu.VMEM((_WIN,), f32),
            pltpu.VMEM((_NCH, 128), i32),
            pltpu.VMEM((_NCH, 128), f32),
            pltpu.VMEM((_NCH, 128), i32),
            pltpu.VMEM((_NCH, 128), f32),
            pltpu.VMEM((_NCH, 128), i32),
            pltpu.VMEM((_NCH, 128), f32),
            pltpu.VMEM((_NB,), i32),
            pltpu.VMEM((_ZB,), f32),
            pltpu.VMEM((16 * _HB,), f32),
            pltpu.VMEM((_HB,), f32),
            pltpu.VMEM((1024,), f32),
            pltpu.VMEM_SHARED((_CLSZ,), f32),
            pltpu.VMEM_SHARED((_MSKZ,), f32),
            pltpu.VMEM_SHARED((16 * _HB,), f32),
        ],
    )
    def k(xt_h, yt_h, a_h, i_h, p_h, mf_h, mm_h, outc_h, outm_h,
          xb, yb, ab, ib, pb, m0b, m1b, ic, vc, i0, v0, i1, v1, bcv, zb,
          hrep, h320, mmv, shc, shm, shh):
        c = lax.axis_index("c")
        s = lax.axis_index("s")
        wid = s * 2 + c

        pltpu.sync_copy(mm_h, mmv)

        def _rowred(r, op):
            acc = mmv[pl.ds(r * 128, 16)]
            for t in range(1, 8):
                acc = op(acc, mmv[pl.ds(r * 128 + t * 16, 16)])
            return acc

        mnx = jnp.min(_rowred(0, jnp.minimum))
        mny = jnp.min(_rowred(1, jnp.minimum))
        mxx = jnp.max(_rowred(2, jnp.maximum))
        mxy = jnp.max(_rowred(3, jnp.maximum))
        dxv = mxx - mnx + jnp.float32(1e-6)
        dyv = mxy - mny + jnp.float32(1e-6)

        def zbody(t, _):
            zb[pl.ds(t * 16, 16)] = jnp.zeros((16,), f32)
            return 0

        lax.fori_loop(0, _ZB // 16, zbody, 0)
        for kk in range(_CPS // _ZB):
            pltpu.sync_copy(zb, shc.at[pl.ds(s * _CPS + kk * _ZB, _ZB)])
        for kk in range(_MPS // _ZB):
            pltpu.sync_copy(zb, shm.at[pl.ds(s * _MPS + kk * _ZB, _ZB)])

        def zh(t, _):
            hrep[pl.ds(t * 16, 16)] = jnp.zeros((16,), f32)
            return 0

        lax.fori_loop(0, 16 * _HB // 16, zh, 0)
        lane320 = lax.iota(i32, 16) * _HB
        hstart = s * _HPW
        for w in range(_HPW // _WIN):
            base = hstart + w * _WIN
            pltpu.sync_copy(a_h.at[pl.ds(base, _WIN)], ab)
            pltpu.sync_copy(p_h.at[pl.ds(base, _WIN)], pb)
            pltpu.sync_copy(i_h.at[pl.ds(base, _WIN)], ib)

            def hbody(t, _):
                o = t * 16
                av = ab[pl.ds(o, 16)]
                pv = pb[pl.ds(o, 16)]
                iv = ib[pl.ds(o, 16)]
                plsc.addupdate_scatter(hrep, [pv * _NB + av + lane320], iv)
                return 0

            lax.fori_loop(0, _WIN // 16, hbody, 0)
        for j in range(_HB // 16):
            acc = hrep[pl.ds(j * 16, 16)]
            for r in range(1, 16):
                acc = acc + hrep[pl.ds(r * _HB + j * 16, 16)]
            h320[pl.ds(j * 16, 16)] = acc
        pltpu.sync_copy(h320, shh.at[pl.ds(s * _HB, _HB)])
        plsc.subcore_barrier()

        pltpu.sync_copy(shh, hrep)
        best = None
        bc_v = None
        for p in range(_NCLS):
            acc = hrep[pl.ds(p * _NB, 16)]
            for r in range(1, 16):
                acc = acc + hrep[pl.ds(r * _HB + p * _NB, 16)]
            if p == 0:
                best = acc
                bc_v = jnp.zeros((16,), i32)
            else:
                upd = acc > best
                best = jnp.where(upd, acc, best)
                bc_v = jnp.where(upd, p, bc_v)
        bcv[...] = bc_v

        start = wid * _PPW
        for w in range(_PPW // _WIN):
            base = start + w * _WIN
            pltpu.sync_copy(xt_h.at[pl.ds(base, _WIN)], xb)
            pltpu.sync_copy(yt_h.at[pl.ds(base, _WIN)], yb)
            pltpu.sync_copy(a_h.at[pl.ds(base, _WIN)], ab)
            pltpu.sync_copy(i_h.at[pl.ds(base, _WIN)], ib)
            pltpu.sync_copy(mf_h.at[0, pl.ds(base, _WIN)], m0b)
            pltpu.sync_copy(mf_h.at[1, pl.ds(base, _WIN)], m1b)

            def fill(t, _):
                for v in range(8):
                    oo = t * 128 + v * 16
                    xv = xb[pl.ds(oo, 16)]
                    yv = yb[pl.ds(oo, 16)]
                    nx = (xv - mnx) / dxv * 256.0
                    ny = (yv - mny) / dyv * 256.0
                    ix = jnp.minimum(jnp.maximum(nx.astype(i32), 0), _RES - 1)
                    iy = jnp.minimum(jnp.maximum(ny.astype(i32), 0), _RES - 1)
                    flat = ix * _RES + iy
                    av = ab[pl.ds(oo, 16)]
                    seg = plsc.load_gather(bcv, [av])
                    iv = ib[pl.ds(oo, 16)]
                    ic[t, pl.ds(v * 16, 16)] = seg * _CELLS + flat
                    vc[t, pl.ds(v * 16, 16)] = iv
                    f2 = flat * 2
                    i0[t, pl.ds(v * 16, 16)] = f2
                    i1[t, pl.ds(v * 16, 16)] = f2 + 1
                    v0[t, pl.ds(v * 16, 16)] = m0b[pl.ds(oo, 16)] * iv
                    v1[t, pl.ds(v * 16, 16)] = m1b[pl.ds(oo, 16)] * iv
                return 0

            lax.fori_loop(0, _NCH, fill, 0)

            def scat(t, _):
                pltpu.sync_copy(vc.at[t], shc.at[ic.at[t]], add=True)
                pltpu.sync_copy(v0.at[t], shm.at[i0.at[t]], add=True)
                pltpu.sync_copy(v1.at[t], shm.at[i1.at[t]], add=True)
                return 0

            lax.fori_loop(0, _NCH, scat, 0)

        plsc.subcore_barrier()
        pltpu.sync_copy(shc.at[pl.ds(s * _CPS, _CPS)],
                        outc_h.at[c, pl.ds(s * _CPS, _CPS)])
        pltpu.sync_copy(shm.at[pl.ds(s * _MPS, _MPS)],
                        outm_h.at[c, pl.ds(s * _MPS, _MPS)])

    return k(xt, yt, asn, ins, pc, mf, mmf)


def _k3_body(cp_r, mp_r, ic_o, mk_o):
    cp = cp_r[...]
    sc = cp[0] + cp[1]
    m = jnp.max(sc, axis=0, keepdims=True)
    io = lax.broadcasted_iota(jnp.int32, sc.shape, 0)
    ic_o[...] = jnp.min(jnp.where(sc == m, io, jnp.int32(2 ** 30)), axis=0)
    mp = mp_r[...]
    mk_o[...] = mp[0] + mp[1]


def _k3(cls_parts, m_parts):
    bc = 4096
    g = _CELLS // bc
    bm = _MSKZ // g
    return pl.pallas_call(
        _k3_body,
        grid=(g,),
        in_specs=[
            pl.BlockSpec((2, _NCLS, bc), lambda i: (0, 0, i)),
            pl.BlockSpec((2, bm), lambda i: (0, i)),
        ],
        out_specs=[
            pl.BlockSpec((bc,), lambda i: (i,)),
            pl.BlockSpec((bm,), lambda i: (i,)),
        ],
        out_shape=[
            jax.ShapeDtypeStruct((_CELLS,), jnp.int32),
            jax.ShapeDtypeStruct((_MSKZ,), jnp.float32),
        ],
    )(cls_parts, m_parts)


def kernel(coords, feats, pseudo_class, boxes, transform,
           W_matte, b_matte, W_feat, b_feat):
    f32 = jnp.float32
    x2 = coords[:, 0].reshape(_ROWS, _COLS)
    y2 = coords[:, 1].reshape(_ROWS, _COLS)
    z2 = coords[:, 2].reshape(_ROWS, _COLS)
    w2 = jnp.concatenate(
        [W_matte.T.astype(f32), W_feat.T.astype(f32),
         jnp.zeros((4, _CIN), f32)], axis=0)
    pc = pseudo_class.astype(jnp.int32)

    xt2, yt2, as2, in2, mf, mm = _k1(
        transform.astype(f32), boxes.astype(f32),
        b_matte.astype(f32), b_feat.astype(f32), x2, y2, z2, feats, w2)
    xt = xt2.reshape(_N)
    yt = yt2.reshape(_N)
    asn = as2.reshape(_N)
    ins = in2.reshape(_N)

    outc, outm = _k2c(xt, yt, asn, ins, pc, mf, mm.reshape(1024))
    img_flat, masks_flat = _k3(outc.reshape(2, _NCLS, _CELLS), outm)
    return (masks_flat.reshape(_RES, _RES, 2),
            img_flat.reshape(_RES, _RES))


# speedup vs baseline: 1.0074x; 1.0074x over previous
import functools

import jax
import jax.numpy as jnp
from jax import lax
from jax.experimental import pallas as pl
from jax.experimental.pallas import tpu as pltpu
from jax.experimental.pallas import tpu_sc as plsc

_N = 131072
_CIN = 64
_RES = 256
_NB = 16
_NCLS = 20

_ROWS = 128
_COLS = 1024
_BR = 8
_BN = _BR * _COLS
_GRID1 = _ROWS // _BR

_NW = 32
_PPW = _N // _NW
_WIN = 2048
_NCH = _WIN // 128

_CELLS = _RES * _RES
_CLSZ = _CELLS * _NCLS
_MSKZ = _CELLS * 2
_CPS = _CLSZ // 16
_MPS = _MSKZ // 16
_ZB = 8192


def _k1_body(t_s, bx_s, bm_s, bf_s, x_r, y_r, z_r, ft_r, w2_r,
             xt_o, yt_o, as_o, in_o, mf_o, mm_o):
    i = pl.program_id(0)
    f32 = jnp.float32
    bf16 = jnp.bfloat16
    x = x_r[...].astype(bf16).astype(f32)
    y = y_r[...].astype(bf16).astype(f32)
    z = z_r[...].astype(bf16).astype(f32)

    def _b(v):
        return v.astype(bf16).astype(f32)

    xt = x * _b(t_s[0, 0]) + y * _b(t_s[0, 1]) + z * _b(t_s[0, 2]) + t_s[0, 3]
    yt = x * _b(t_s[1, 0]) + y * _b(t_s[1, 1]) + z * _b(t_s[1, 2]) + t_s[1, 3]
    zt = x * _b(t_s[2, 0]) + y * _b(t_s[2, 1]) + z * _b(t_s[2, 2]) + t_s[2, 3]

    best = None
    asn = None
    ins = None
    for j in range(_NB):
        dx = xt - bx_s[j, 0]
        dy = yt - bx_s[j, 1]
        dz = zt - bx_s[j, 2]
        d = jnp.sqrt(dx * dx + dy * dy + dz * dz)
        hx = jnp.abs(bx_s[j, 3]) + 1e-3
        hy = jnp.abs(bx_s[j, 4]) + 1e-3
        hz = jnp.abs(bx_s[j, 5]) + 1e-3
        inj = jnp.where(
            (jnp.abs(dx) <= hx) & (jnp.abs(dy) <= hy) & (jnp.abs(dz) <= hz),
            jnp.float32(1.0), jnp.float32(0.0))
        if j == 0:
            best = d
            asn = jnp.zeros(d.shape, jnp.int32)
            ins = inj
        else:
            upd = d < best
            best = jnp.where(upd, d, best)
            asn = jnp.where(upd, j, asn)
            ins = jnp.where(upd, inj, ins)
    xt_o[...] = xt
    yt_o[...] = yt
    as_o[...] = asn
    in_o[...] = ins

    fb = ft_r[...].astype(bf16)
    res = lax.dot_general(w2_r[...].astype(bf16), fb, (((1,), (1,)), ((), ())),
                          preferred_element_type=jnp.float32)
    a0 = jax.nn.sigmoid(res[0:1, :] + bm_s[0])
    a1 = jax.nn.sigmoid(res[1:2, :] + bm_s[1])
    mf_o[0:1, :] = (res[2:3, :] + bf_s[0]) * a0
    mf_o[1:2, :] = (res[3:4, :] + bf_s[1]) * a1

    @pl.when(i == 0)
    def _init():
        mm_o[...] = jnp.zeros((8, 128), jnp.float32)
        mm_o[0:2, :] = jnp.full((2, 128), jnp.inf, jnp.float32)
        mm_o[2:4, :] = jnp.full((2, 128), -jnp.inf, jnp.float32)

    mm_o[0:1, :] = jnp.minimum(mm_o[0:1, :], jnp.min(xt))
    mm_o[1:2, :] = jnp.minimum(mm_o[1:2, :], jnp.min(yt))
    mm_o[2:3, :] = jnp.maximum(mm_o[2:3, :], jnp.max(xt))
    mm_o[3:4, :] = jnp.maximum(mm_o[3:4, :], jnp.max(yt))


def _k1(transform, boxes, b_matte, b_feat, x2, y2, z2, ftT, w2):
    f32 = jnp.float32
    return pl.pallas_call(
        _k1_body,
        grid=(_GRID1,),
        in_specs=[
            pl.BlockSpec(memory_space=pltpu.SMEM),
            pl.BlockSpec(memory_space=pltpu.SMEM),
            pl.BlockSpec(memory_space=pltpu.SMEM),
            pl.BlockSpec(memory_space=pltpu.SMEM),
            pl.BlockSpec((_BR, _COLS), lambda i: (i, 0)),
            pl.BlockSpec((_BR, _COLS), lambda i: (i, 0)),
            pl.BlockSpec((_BR, _COLS), lambda i: (i, 0)),
            pl.BlockSpec((_BN, _CIN), lambda i: (i, 0)),
            pl.BlockSpec((8, _CIN), lambda i: (0, 0)),
        ],
        out_specs=[
            pl.BlockSpec((_BR, _COLS), lambda i: (i, 0)),
            pl.BlockSpec((_BR, _COLS), lambda i: (i, 0)),
            pl.BlockSpec((_BR, _COLS), lambda i: (i, 0)),
            pl.BlockSpec((_BR, _COLS), lambda i: (i, 0)),
            pl.BlockSpec((2, _BN), lambda i: (0, i)),
            pl.BlockSpec((8, 128), lambda i: (0, 0)),
        ],
        out_shape=[
            jax.ShapeDtypeStruct((_ROWS, _COLS), f32),
            jax.ShapeDtypeStruct((_ROWS, _COLS), f32),
            jax.ShapeDtypeStruct((_ROWS, _COLS), jnp.int32),
            jax.ShapeDtypeStruct((_ROWS, _COLS), f32),
            jax.ShapeDtypeStruct((2, _N), f32),
            jax.ShapeDtypeStruct((8, 128), f32),
        ],
    )(transform, boxes, b_matte, b_feat, x2, y2, z2, ftT, w2)


def _sc_mesh():
    return plsc.VectorSubcoreMesh(core_axis_name="c", subcore_axis_name="s")


_SC_PARAMS = pltpu.CompilerParams(needs_layout_passes=False)


def _k2a(asn, pc, ins):
    @functools.partial(
        pl.kernel,
        mesh=_sc_mesh(),
        out_type=jax.ShapeDtypeStruct((_NW, _NCLS * _NB * 16), jnp.float32),
        compiler_params=_SC_PARAMS,
        scratch_types=[
            pltpu.VMEM((_WIN,), jnp.int32),
            pltpu.VMEM((_WIN,), jnp.int32),
            pltpu.VMEM((_WIN,), jnp.float32),
            pltpu.VMEM((_NCLS * _NB * 16,), jnp.float32),
        ],
    )
    def k(a_h, p_h, i_h, out_h, ab, pb, ib, hist):
        c = lax.axis_index("c")
        s = lax.axis_index("s")
        wid = s * 2 + c

        def zbody(t, _):
            hist[pl.ds(t * 16, 16)] = jnp.zeros((16,), jnp.float32)
            return 0

        lax.fori_loop(0, _NCLS * _NB, zbody, 0)
        lane = lax.iota(jnp.int32, 16)
        start = wid * _PPW
        for w in range(_PPW // _WIN):
            base = start + w * _WIN
            pltpu.sync_copy(a_h.at[pl.ds(base, _WIN)], ab)
            pltpu.sync_copy(p_h.at[pl.ds(base, _WIN)], pb)
            pltpu.sync_copy(i_h.at[pl.ds(base, _WIN)], ib)

            def body(t, _):
                o = t * 16
                av = ab[pl.ds(o, 16)]
                pv = pb[pl.ds(o, 16)]
                iv = ib[pl.ds(o, 16)]
                idx = (pv * _NB + av) * 16 + lane
                plsc.addupdate_scatter(hist, [idx], iv)
                return 0

            lax.fori_loop(0, _WIN // 16, body, 0)
        pltpu.sync_copy(hist, out_h.at[wid])

    return k(asn, pc, ins)


def _k2b_body(hp_r, mm_r, bc_o, pr_o):
    h = jnp.sum(jnp.sum(hp_r[...], axis=3), axis=0)
    m = jnp.max(h, axis=0, keepdims=True)
    io = lax.broadcasted_iota(jnp.int32, (_NCLS, _NB), 0)
    bc = jnp.min(jnp.where(h == m, io, jnp.int32(2 ** 30)), axis=0)
    bc_o[...] = bc.reshape(1, _NB)
    mm = mm_r[...]
    mnx = jnp.min(mm[0:1, :])
    mny = jnp.min(mm[1:2, :])
    mxx = jnp.max(mm[2:3, :])
    mxy = jnp.max(mm[3:4, :])
    dxv = mxx - mnx + jnp.float32(1e-6)
    dyv = mxy - mny + jnp.float32(1e-6)
    io2 = lax.broadcasted_iota(jnp.int32, (1, _NB), 1)
    p = jnp.where(io2 == 0, mnx,
                  jnp.where(io2 == 1, mny,
                            jnp.where(io2 == 2, dxv,
                                      jnp.where(io2 == 3, dyv, 0.0))))
    pr_o[...] = p.astype(jnp.float32)


def _k2b(hp, mm):
    return pl.pallas_call(
        _k2b_body,
        out_shape=[
            jax.ShapeDtypeStruct((1, _NB), jnp.int32),
            jax.ShapeDtypeStruct((1, _NB), jnp.float32),
        ],
    )(hp, mm)


def _k2c(xt, yt, asn, ins, mf, bc, prm):
    f32 = jnp.float32
    i32 = jnp.int32

    @functools.partial(
        pl.kernel,
        mesh=_sc_mesh(),
        out_type=[
            jax.ShapeDtypeStruct((2, _CLSZ), f32),
            jax.ShapeDtypeStruct((2, _MSKZ), f32),
        ],
        compiler_params=_SC_PARAMS,
        scratch_types=[
            pltpu.VMEM((_WIN,), f32),
            pltpu.VMEM((_WIN,), f32),
            pltpu.VMEM((_WIN,), i32),
            pltpu.VMEM((_WIN,), f32),
            pltpu.VMEM((_WIN,), f32),
            pltpu.VMEM((_WIN,), f32),
            pltpu.VMEM((_NCH, 128), i32),
            pltpu.VMEM((_NCH, 128), f32),
            pltpu.VMEM((_NCH, 128), i32),
            pltpu.VMEM((_NCH, 128), f32),
            pltpu.VMEM((_NCH, 128), i32),
            pltpu.VMEM((_NCH, 128), f32),
            pltpu.VMEM((_NB,), i32),
            pltpu.VMEM((_NB,), f32),
            pltpu.VMEM((_ZB,), f32),
            pltpu.VMEM_SHARED((_CLSZ,), f32),
            pltpu.VMEM_SHARED((_MSKZ,), f32),
        ],
    )
    def k(xt_h, yt_h, a_h, i_h, mf_h, bc_h, pr_h, outc_h, outm_h,
          xb, yb, ab, ib, m0b, m1b, ic, vc, i0, v0, i1, v1, bcv, prv, zb,
          shc, shm):
        c = lax.axis_index("c")
        s = lax.axis_index("s")
        wid = s * 2 + c
        pltpu.sync_copy(bc_h, bcv)
        pltpu.sync_copy(pr_h, prv)
        prvv = prv[...]
        mnx = prvv[0]
        mny = prvv[1]
        dxv = prvv[2]
        dyv = prvv[3]

        def zbody(t, _):
            zb[pl.ds(t * 16, 16)] = jnp.zeros((16,), f32)
            return 0

        lax.fori_loop(0, _ZB // 16, zbody, 0)
        for kk in range(_CPS // _ZB):
            pltpu.sync_copy(zb, shc.at[pl.ds(s * _CPS + kk * _ZB, _ZB)])
        pltpu.sync_copy(zb, shm.at[pl.ds(s * _MPS, _MPS)])
        plsc.subcore_barrier()

        start = wid * _PPW
        for w in range(_PPW // _WIN):
            base = start + w * _WIN
            pltpu.sync_copy(xt_h.at[pl.ds(base, _WIN)], xb)
            pltpu.sync_copy(yt_h.at[pl.ds(base, _WIN)], yb)
            pltpu.sync_copy(a_h.at[pl.ds(base, _WIN)], ab)
            pltpu.sync_copy(i_h.at[pl.ds(base, _WIN)], ib)
            pltpu.sync_copy(mf_h.at[0, pl.ds(base, _WIN)], m0b)
            pltpu.sync_copy(mf_h.at[1, pl.ds(base, _WIN)], m1b)

            def fill(t, _):
                for v in range(8):
                    oo = t * 128 + v * 16
                    xv = xb[pl.ds(oo, 16)]
                    yv = yb[pl.ds(oo, 16)]
                    nx = (xv - mnx) / dxv * 256.0
                    ny = (yv - mny) / dyv * 256.0
                    ix = jnp.minimum(jnp.maximum(nx.astype(i32), 0), _RES - 1)
                    iy = jnp.minimum(jnp.maximum(ny.astype(i32), 0), _RES - 1)
                    flat = ix * _RES + iy
                    av = ab[pl.ds(oo, 16)]
                    seg = plsc.load_gather(bcv, [av])
                    iv = ib[pl.ds(oo, 16)]
                    ic[t, pl.ds(v * 16, 16)] = seg * _CELLS + flat
                    vc[t, pl.ds(v * 16, 16)] = iv
                    f2 = flat * 2
                    i0[t, pl.ds(v * 16, 16)] = f2
                    i1[t, pl.ds(v * 16, 16)] = f2 + 1
                    v0[t, pl.ds(v * 16, 16)] = m0b[pl.ds(oo, 16)] * iv
                    v1[t, pl.ds(v * 16, 16)] = m1b[pl.ds(oo, 16)] * iv
                return 0

            lax.fori_loop(0, _NCH, fill, 0)

            def scat(t, _):
                pltpu.sync_copy(vc.at[t], shc.at[ic.at[t]], add=True)
                pltpu.sync_copy(v0.at[t], shm.at[i0.at[t]], add=True)
                pltpu.sync_copy(v1.at[t], shm.at[i1.at[t]], add=True)
                return 0

            lax.fori_loop(0, _NCH, scat, 0)

        plsc.subcore_barrier()
        pltpu.sync_copy(shc.at[pl.ds(s * _CPS, _CPS)],
                        outc_h.at[c, pl.ds(s * _CPS, _CPS)])
        pltpu.sync_copy(shm.at[pl.ds(s * _MPS, _MPS)],
                        outm_h.at[c, pl.ds(s * _MPS, _MPS)])

    return k(xt, yt, asn, ins, mf, bc, prm)


def _k3_body(cp_r, mp_r, ic_o, mk_o):
    cp = cp_r[...]
    sc = cp[0] + cp[1]
    m = jnp.max(sc, axis=0, keepdims=True)
    io = lax.broadcasted_iota(jnp.int32, sc.shape, 0)
    ic_o[...] = jnp.min(jnp.where(sc == m, io, jnp.int32(2 ** 30)), axis=0)
    mp = mp_r[...]
    mk_o[...] = mp[0] + mp[1]


def _k3(cls_parts, m_parts):
    bc = 4096
    g = _CELLS // bc
    bm = _MSKZ // g
    return pl.pallas_call(
        _k3_body,
        grid=(g,),
        in_specs=[
            pl.BlockSpec((2, _NCLS, bc), lambda i: (0, 0, i)),
            pl.BlockSpec((2, bm), lambda i: (0, i)),
        ],
        out_specs=[
            pl.BlockSpec((bc,), lambda i: (i,)),
            pl.BlockSpec((bm,), lambda i: (i,)),
        ],
        out_shape=[
            jax.ShapeDtypeStruct((_CELLS,), jnp.int32),
            jax.ShapeDtypeStruct((_MSKZ,), jnp.float32),
        ],
    )(cls_parts, m_parts)


def kernel(coords, feats, pseudo_class, boxes, transform,
           W_matte, b_matte, W_feat, b_feat):
    f32 = jnp.float32
    x2 = coords[:, 0].reshape(_ROWS, _COLS)
    y2 = coords[:, 1].reshape(_ROWS, _COLS)
    z2 = coords[:, 2].reshape(_ROWS, _COLS)
    w2 = jnp.concatenate(
        [W_matte.T.astype(f32), W_feat.T.astype(f32),
         jnp.zeros((4, _CIN), f32)], axis=0)
    pc = pseudo_class.astype(jnp.int32)

    xt2, yt2, as2, in2, mf, mm = _k1(
        transform.astype(f32), boxes.astype(f32),
        b_matte.astype(f32), b_feat.astype(f32), x2, y2, z2, feats, w2)
    xt = xt2.reshape(_N)
    yt = yt2.reshape(_N)
    asn = as2.reshape(_N)
    ins = in2.reshape(_N)

    hp = _k2a(asn, pc, ins).reshape(_NW, _NCLS, _NB, 16)
    bc, prm = _k2b(hp, mm)
    outc, outm = _k2c(xt, yt, asn, ins, mf, bc.reshape(_NB), prm.reshape(_NB))
    img_flat, masks_flat = _k3(outc.reshape(2, _NCLS, _CELLS), outm)
    return (masks_flat.reshape(_RES, _RES, 2),
            img_flat.reshape(_RES, _RES))
